# baseline (speedup 1.0000x reference)
import functools

import jax
import jax.numpy as jnp
from jax import lax
from jax.experimental import pallas as pl
from jax.experimental.pallas import tpu as pltpu
from jax.experimental.pallas import tpu_sc as plsc

N_B, N_TOKENS, N_T, N_C = 8, 675, 16, 256
N_BOT = 135
T_BLK = 8
IDX_PAD = 144
N_ROWS = N_B * N_T
_SQRT_HALF = 0.7071067811865476


def _gelu_pre(m):
    return (_SQRT_HALF * m) * (1.0 + jax.lax.erf(m))


def _score_body(x_ref, W1_ref, b1_ref, W2_ref, b2_ref, W3_ref, b3_ref,
                w4d_ref, tri_ref, idx_ref, sr_ref):
    X = x_ref[0]
    NT = N_TOKENS * T_BLK
    Xf = X.reshape(NT, N_C)

    mu = jnp.mean(Xf, axis=-1, keepdims=True)
    xc = Xf - mu
    var = jnp.mean(xc * xc, axis=-1, keepdims=True)
    ln = xc * jax.lax.rsqrt(var + 1e-5)
    u = _gelu_pre(jnp.dot(ln, W1_ref[...], preferred_element_type=jnp.float32)
                  + b1_ref[...])
    u3 = u.reshape(N_TOKENS, T_BLK, N_C)
    glob = jnp.sum(u3[:, :, N_C // 2:], axis=0) / float(N_TOKENS)
    H = jnp.concatenate(
        [u3[:, :, :N_C // 2],
         jnp.broadcast_to(glob[None], (N_TOKENS, T_BLK, N_C // 2))], axis=-1)
    h2 = _gelu_pre(jnp.dot(H.reshape(NT, N_C), W2_ref[...],
                           preferred_element_type=jnp.float32) + b2_ref[...])
    h3 = _gelu_pre(jnp.dot(h2, W3_ref[...],
                           preferred_element_type=jnp.float32) + b3_ref[...])
    d = jnp.sum(h3 * w4d_ref[...], axis=-1).reshape(N_TOKENS, T_BLK)
    dt = d.T

    k = jax.lax.bitcast_convert_type(dt, jnp.int32)
    key = k ^ ((k >> 31) & jnp.int32(0x7FFFFFFF))
    ukey = key ^ jnp.int32(-2147483648)
    active = jnp.ones((T_BLK, N_TOKENS), jnp.int32)
    bottom = jnp.zeros((T_BLK, N_TOKENS), jnp.int32)
    need = jnp.full((T_BLK, 1), N_BOT, jnp.int32)
    for bit in range(31, -1, -1):
        bitv = (ukey >> bit) & 1
        zeros = active * (bitv ^ 1)
        nz = jnp.sum(zeros, axis=1, keepdims=True)
        go_zero = nz >= need
        bottom = jnp.where(go_zero, bottom, bottom + zeros)
        need = jnp.where(go_zero, need, need - nz)
        active = jnp.where(go_zero, zeros, active * bitv)
    idx = jax.lax.broadcasted_iota(jnp.int32, (T_BLK, N_TOKENS), 1)
    for bit in range(9, -1, -1):
        bitv = (idx >> bit) & 1
        ones = active * bitv
        n1 = jnp.sum(ones, axis=1, keepdims=True)
        go_one = n1 >= need
        bottom = jnp.where(go_one, bottom, bottom + ones)
        need = jnp.where(go_one, need, need - n1)
        active = jnp.where(go_one, ones, active * (bitv ^ 1))
    bottom = bottom + active * jnp.where(need > 0, 1, 0)

    bot_f = bottom.astype(jnp.float32)
    csum = jnp.dot(bot_f, tri_ref[...],
                   preferred_element_type=jnp.float32)
    kk = jax.lax.broadcasted_iota(jnp.int32, (T_BLK, N_TOKENS, N_BOT), 2)
    nn = jax.lax.broadcasted_iota(jnp.int32, (T_BLK, N_TOKENS, N_BOT), 1)
    csi = csum.astype(jnp.int32)[:, :, None]
    oh = ((csi == kk + 1) & (bottom[:, :, None] == 1)).astype(jnp.int32)
    tok = jnp.sum(oh * nn, axis=1)
    b = pl.program_id(0)
    tt = pl.program_id(1)
    rowc = (b * (N_TOKENS * N_T) + tt * T_BLK
            + jax.lax.broadcasted_iota(jnp.int32, (T_BLK, N_BOT), 0))
    flat = tok * N_T + rowc
    idx_ref[0] = jnp.concatenate(
        [flat, jnp.zeros((T_BLK, IDX_PAD - N_BOT), jnp.int32)], axis=1)
    sr_ref[0] = jnp.sum(X, axis=0)


_ROWS_PER_TILE = N_ROWS // 32


def _sc_gather_sum(idx2, xt):
    mesh = plsc.VectorSubcoreMesh(core_axis_name="c", subcore_axis_name="s")

    @functools.partial(
        pl.kernel, mesh=mesh,
        out_type=jax.ShapeDtypeStruct((N_ROWS, N_C), jnp.float32),
        scratch_types=[
            pltpu.VMEM((128,), jnp.int32),
            pltpu.VMEM((8,), jnp.int32),
            pltpu.VMEM((128, N_C), jnp.float32),
            pltpu.VMEM((8, N_C), jnp.float32),
            pltpu.VMEM((N_C,), jnp.float32),
            pltpu.SemaphoreType.DMA,
        ],
    )
    def k(idx_hbm, xt_hbm, out_hbm, idxa_v, idxb_v, rows_a, rows_b,
          acc_v, sem):
        wid = lax.axis_index("s") * 2 + lax.axis_index("c")
        for rr in range(_ROWS_PER_TILE):
            r = wid * _ROWS_PER_TILE + rr
            pltpu.sync_copy(idx_hbm.at[r, pl.ds(0, 128)], idxa_v)
            pltpu.sync_copy(idx_hbm.at[r, pl.ds(128, 8)], idxb_v)
            pltpu.async_copy(xt_hbm.at[idxa_v], rows_a, sem).wait()
            pltpu.async_copy(xt_hbm.at[idxb_v], rows_b, sem).wait()

            def accum_a(j, acc):
                return tuple(acc[c] + rows_a[j, pl.ds(c * 16, 16)]
                             for c in range(N_C // 16))

            acc = tuple(jnp.zeros((16,), jnp.float32)
                        for _ in range(N_C // 16))
            acc = lax.fori_loop(0, 128, accum_a, acc, unroll=False)

            def accum_b(j, acc):
                return tuple(acc[c] + rows_b[j, pl.ds(c * 16, 16)]
                             for c in range(N_C // 16))

            acc = lax.fori_loop(0, N_BOT - 128, accum_b, acc, unroll=False)
            for c in range(N_C // 16):
                acc_v[pl.ds(c * 16, 16)] = acc[c]
            pltpu.sync_copy(acc_v, out_hbm.at[r])

    return k(idx2, xt)


def _final_body(x_ref, sr_ref, bs_ref, wl_ref, o_ref):
    X = x_ref[0]
    wl2 = wl_ref[...]
    S = sr_ref[0] * wl2
    pooled = S / float(N_TOKENS)
    add2 = pooled + (S * (1.0 + wl2) - bs_ref[0] * wl2
                     - float(N_BOT) * pooled * wl2) / float(N_TOKENS - N_BOT)
    o_ref[0] = X + add2[None]


def kernel(x, wl, ln_g, ln_b, W1, b1, W2, b2, W3, b3, W4, b4):
    wl2 = wl.reshape(1, N_C)
    W1s = (ln_g[:, None] * W1) * _SQRT_HALF
    b1r = ((ln_b @ W1 + b1) * _SQRT_HALF).reshape(1, N_C)
    W2s = W2 * _SQRT_HALF
    b2r = (b2 * _SQRT_HALF).reshape(1, N_C // 2)
    W3s = W3 * _SQRT_HALF
    b3r = (b3 * _SQRT_HALF).reshape(1, N_C // 4)
    w4d = (W4[:, 0] - W4[:, 1]).reshape(1, N_C // 4)

    grid = (N_B, N_T // T_BLK)
    blk = pl.BlockSpec((1, N_TOKENS, T_BLK, N_C), lambda b, t: (b, 0, t, 0))
    small = lambda s: pl.BlockSpec(s, lambda b, t: (0,) * len(s))
    row_blk = lambda w: pl.BlockSpec((1, T_BLK, w), lambda b, t: (b, t, 0))

    ri = jnp.arange(N_TOKENS, dtype=jnp.int32)
    tri = (ri[:, None] <= ri[None, :]).astype(jnp.float32)

    idxs, sr = pl.pallas_call(
        _score_body,
        grid=grid,
        in_specs=[
            blk,
            small((N_C, N_C)), small((1, N_C)),
            small((N_C, N_C // 2)), small((1, N_C // 2)),
            small((N_C // 2, N_C // 4)), small((1, N_C // 4)),
            small((1, N_C // 4)),
            small((N_TOKENS, N_TOKENS)),
        ],
        out_specs=[row_blk(IDX_PAD), row_blk(N_C)],
        out_shape=[
            jax.ShapeDtypeStruct((N_B, N_T, IDX_PAD), jnp.int32),
            jax.ShapeDtypeStruct((N_B, N_T, N_C), jnp.float32),
        ],
        compiler_params=pltpu.CompilerParams(
            dimension_semantics=("parallel", "parallel")),
    )(x, W1s, b1r, W2s, b2r, W3s, b3r, w4d, tri)

    botsum = _sc_gather_sum(idxs.reshape(N_ROWS, IDX_PAD),
                            x.reshape(N_B * N_TOKENS * N_T, N_C))

    out = pl.pallas_call(
        _final_body,
        grid=grid,
        in_specs=[blk, row_blk(N_C), row_blk(N_C), small((1, N_C))],
        out_specs=blk,
        out_shape=jax.ShapeDtypeStruct((N_B, N_TOKENS, N_T, N_C), jnp.float32),
        compiler_params=pltpu.CompilerParams(
            dimension_semantics=("parallel", "parallel")),
    )(x, sr, botsum.reshape(N_B, N_T, N_C), wl2)
    return out


# speedup vs baseline: 1.0289x; 1.0289x over previous
import functools

import jax
import jax.numpy as jnp
from jax import lax
from jax.experimental import pallas as pl
from jax.experimental.pallas import tpu as pltpu
from jax.experimental.pallas import tpu_sc as plsc

N_B, N_TOKENS, N_T, N_C = 8, 675, 16, 256
N_BOT = 135
T_BLK = 8
IDX_PAD = 144
N_ROWS = N_B * N_T
_SQRT_HALF = 0.7071067811865476


def _gelu_pre(m):
    return (_SQRT_HALF * m) * (1.0 + jax.lax.erf(m))


def _score_body(x_ref, W1_ref, b1_ref, W2_ref, b2_ref, W3_ref, b3_ref,
                w4d_ref, tri_ref, idx_ref, sr_ref):
    X = x_ref[0]
    NT = N_TOKENS * T_BLK
    Xf = X.reshape(NT, N_C)

    mu = jnp.mean(Xf, axis=-1, keepdims=True)
    xc = Xf - mu
    var = jnp.mean(xc * xc, axis=-1, keepdims=True)
    ln = xc * jax.lax.rsqrt(var + 1e-5)
    u = _gelu_pre(jnp.dot(ln, W1_ref[...], preferred_element_type=jnp.float32)
                  + b1_ref[...])
    u3 = u.reshape(N_TOKENS, T_BLK, N_C)
    glob = jnp.sum(u3[:, :, N_C // 2:], axis=0) / float(N_TOKENS)
    H = jnp.concatenate(
        [u3[:, :, :N_C // 2],
         jnp.broadcast_to(glob[None], (N_TOKENS, T_BLK, N_C // 2))], axis=-1)
    h2 = _gelu_pre(jnp.dot(H.reshape(NT, N_C), W2_ref[...],
                           preferred_element_type=jnp.float32) + b2_ref[...])
    h3 = _gelu_pre(jnp.dot(h2, W3_ref[...],
                           preferred_element_type=jnp.float32) + b3_ref[...])
    d = jnp.sum(h3 * w4d_ref[...], axis=-1).reshape(N_TOKENS, T_BLK)
    dt = d.T

    k = jax.lax.bitcast_convert_type(dt, jnp.int32)
    key = k ^ ((k >> 31) & jnp.int32(0x7FFFFFFF))
    ukey = key ^ jnp.int32(-2147483648)
    active = jnp.ones((T_BLK, N_TOKENS), jnp.int32)
    bottom = jnp.zeros((T_BLK, N_TOKENS), jnp.int32)
    need = jnp.full((T_BLK, 1), N_BOT, jnp.int32)
    for bit in range(31, -1, -1):
        bitv = (ukey >> bit) & 1
        zeros = active * (bitv ^ 1)
        nz = jnp.sum(zeros, axis=1, keepdims=True)
        go_zero = nz >= need
        bottom = jnp.where(go_zero, bottom, bottom + zeros)
        need = jnp.where(go_zero, need, need - nz)
        active = jnp.where(go_zero, zeros, active * bitv)
    idx = jax.lax.broadcasted_iota(jnp.int32, (T_BLK, N_TOKENS), 1)
    for bit in range(9, -1, -1):
        bitv = (idx >> bit) & 1
        ones = active * bitv
        n1 = jnp.sum(ones, axis=1, keepdims=True)
        go_one = n1 >= need
        bottom = jnp.where(go_one, bottom, bottom + ones)
        need = jnp.where(go_one, need, need - n1)
        active = jnp.where(go_one, ones, active * (bitv ^ 1))
    bottom = bottom + active * jnp.where(need > 0, 1, 0)

    bot_f = bottom.astype(jnp.float32)
    csum = jnp.dot(bot_f, tri_ref[...],
                   preferred_element_type=jnp.float32)
    kk = jax.lax.broadcasted_iota(jnp.int32, (T_BLK, N_TOKENS, N_BOT), 2)
    nn = jax.lax.broadcasted_iota(jnp.int32, (T_BLK, N_TOKENS, N_BOT), 1)
    csi = csum.astype(jnp.int32)[:, :, None]
    oh = ((csi == kk + 1) & (bottom[:, :, None] == 1)).astype(jnp.int32)
    tok = jnp.sum(oh * nn, axis=1)
    b = pl.program_id(0)
    tt = pl.program_id(1)
    rowc = (b * (N_TOKENS * N_T) + tt * T_BLK
            + jax.lax.broadcasted_iota(jnp.int32, (T_BLK, N_BOT), 0))
    flat = tok * N_T + rowc
    idx_ref[0] = jnp.concatenate(
        [flat, jnp.zeros((T_BLK, IDX_PAD - N_BOT), jnp.int32)], axis=1)
    sr_ref[0] = jnp.sum(X, axis=0)


_ROWS_PER_TILE = N_ROWS // 32


def _sc_gather_sum(idx2, xt):
    mesh = plsc.VectorSubcoreMesh(core_axis_name="c", subcore_axis_name="s")

    @functools.partial(
        pl.kernel, mesh=mesh,
        out_type=jax.ShapeDtypeStruct((N_ROWS, N_C), jnp.float32),
        scratch_types=[
            pltpu.VMEM((2, 128), jnp.int32),
            pltpu.VMEM((2, 8), jnp.int32),
            pltpu.VMEM((2, 128, N_C), jnp.float32),
            pltpu.VMEM((2, 8, N_C), jnp.float32),
            pltpu.VMEM((N_C,), jnp.float32),
            pltpu.SemaphoreType.DMA,
            pltpu.SemaphoreType.DMA,
        ],
    )
    def k(idx_hbm, xt_hbm, out_hbm, idxa_v, idxb_v, rows_a, rows_b,
          acc_v, sem0, sem1):
        wid = lax.axis_index("s") * 2 + lax.axis_index("c")
        sems = (sem0, sem1)
        pend = [None, None]

        def issue(rr, slot):
            r = wid * _ROWS_PER_TILE + rr
            pltpu.sync_copy(idx_hbm.at[r, pl.ds(0, 128)], idxa_v.at[slot])
            pltpu.sync_copy(idx_hbm.at[r, pl.ds(128, 8)], idxb_v.at[slot])
            ca = pltpu.async_copy(xt_hbm.at[idxa_v.at[slot]],
                                  rows_a.at[slot], sems[slot])
            cb = pltpu.async_copy(xt_hbm.at[idxb_v.at[slot]],
                                  rows_b.at[slot], sems[slot])
            pend[slot] = (ca, cb)

        issue(0, 0)
        for rr in range(_ROWS_PER_TILE):
            slot = rr % 2
            if rr + 1 < _ROWS_PER_TILE:
                issue(rr + 1, (rr + 1) % 2)
            ca, cb = pend[slot]
            ca.wait()
            cb.wait()

            def accum_a(j, acc, slot=slot):
                return tuple(acc[c] + rows_a[slot, j, pl.ds(c * 16, 16)]
                             for c in range(N_C // 16))

            acc = tuple(jnp.zeros((16,), jnp.float32)
                        for _ in range(N_C // 16))
            acc = lax.fori_loop(0, 128, accum_a, acc, unroll=False)

            def accum_b(j, acc, slot=slot):
                return tuple(acc[c] + rows_b[slot, j, pl.ds(c * 16, 16)]
                             for c in range(N_C // 16))

            acc = lax.fori_loop(0, N_BOT - 128, accum_b, acc, unroll=False)
            for c in range(N_C // 16):
                acc_v[pl.ds(c * 16, 16)] = acc[c]
            pltpu.sync_copy(acc_v, out_hbm.at[wid * _ROWS_PER_TILE + rr])

    return k(idx2, xt)


def _final_body(x_ref, sr_ref, bs_ref, wl_ref, o_ref):
    X = x_ref[0]
    wl2 = wl_ref[...]
    S = sr_ref[0] * wl2
    pooled = S / float(N_TOKENS)
    add2 = pooled + (S * (1.0 + wl2) - bs_ref[0] * wl2
                     - float(N_BOT) * pooled * wl2) / float(N_TOKENS - N_BOT)
    o_ref[0] = X + add2[None]


def kernel(x, wl, ln_g, ln_b, W1, b1, W2, b2, W3, b3, W4, b4):
    wl2 = wl.reshape(1, N_C)
    W1s = (ln_g[:, None] * W1) * _SQRT_HALF
    b1r = ((ln_b @ W1 + b1) * _SQRT_HALF).reshape(1, N_C)
    W2s = W2 * _SQRT_HALF
    b2r = (b2 * _SQRT_HALF).reshape(1, N_C // 2)
    W3s = W3 * _SQRT_HALF
    b3r = (b3 * _SQRT_HALF).reshape(1, N_C // 4)
    w4d = (W4[:, 0] - W4[:, 1]).reshape(1, N_C // 4)

    grid = (N_B, N_T // T_BLK)
    blk = pl.BlockSpec((1, N_TOKENS, T_BLK, N_C), lambda b, t: (b, 0, t, 0))
    small = lambda s: pl.BlockSpec(s, lambda b, t: (0,) * len(s))
    row_blk = lambda w: pl.BlockSpec((1, T_BLK, w), lambda b, t: (b, t, 0))

    ri = jnp.arange(N_TOKENS, dtype=jnp.int32)
    tri = (ri[:, None] <= ri[None, :]).astype(jnp.float32)

    idxs, sr = pl.pallas_call(
        _score_body,
        grid=grid,
        in_specs=[
            blk,
            small((N_C, N_C)), small((1, N_C)),
            small((N_C, N_C // 2)), small((1, N_C // 2)),
            small((N_C // 2, N_C // 4)), small((1, N_C // 4)),
            small((1, N_C // 4)),
            small((N_TOKENS, N_TOKENS)),
        ],
        out_specs=[row_blk(IDX_PAD), row_blk(N_C)],
        out_shape=[
            jax.ShapeDtypeStruct((N_B, N_T, IDX_PAD), jnp.int32),
            jax.ShapeDtypeStruct((N_B, N_T, N_C), jnp.float32),
        ],
        compiler_params=pltpu.CompilerParams(
            dimension_semantics=("parallel", "parallel")),
    )(x, W1s, b1r, W2s, b2r, W3s, b3r, w4d, tri)

    botsum = _sc_gather_sum(idxs.reshape(N_ROWS, IDX_PAD),
                            x.reshape(N_B * N_TOKENS * N_T, N_C))

    out = pl.pallas_call(
        _final_body,
        grid=grid,
        in_specs=[blk, row_blk(N_C), row_blk(N_C), small((1, N_C))],
        out_specs=blk,
        out_shape=jax.ShapeDtypeStruct((N_B, N_TOKENS, N_T, N_C), jnp.float32),
        compiler_params=pltpu.CompilerParams(
            dimension_semantics=("parallel", "parallel")),
    )(x, sr, botsum.reshape(N_B, N_T, N_C), wl2)
    return out


# speedup vs baseline: 1.0377x; 1.0085x over previous
import functools

import jax
import jax.numpy as jnp
from jax import lax
from jax.experimental import pallas as pl
from jax.experimental.pallas import tpu as pltpu
from jax.experimental.pallas import tpu_sc as plsc

N_B, N_TOKENS, N_T, N_C = 8, 675, 16, 256
N_BOT = 135
T_BLK = 8
IDX_PAD = 144
N_ROWS = N_B * N_T
_SQRT_HALF = 0.7071067811865476


def _gelu_pre(m):
    return (_SQRT_HALF * m) * (1.0 + jax.lax.erf(m))


def _score_body(x_ref, W1_ref, b1_ref, W2_ref, b2_ref, W3_ref, b3_ref,
                w4d_ref, tri_ref, idx_ref, sr_ref):
    X = x_ref[0]
    NT = N_TOKENS * T_BLK
    Xf = X.reshape(NT, N_C)

    mu = jnp.mean(Xf, axis=-1, keepdims=True)
    xc = Xf - mu
    var = jnp.mean(xc * xc, axis=-1, keepdims=True)
    ln = xc * jax.lax.rsqrt(var + 1e-5)
    u = _gelu_pre(jnp.dot(ln, W1_ref[...], preferred_element_type=jnp.float32)
                  + b1_ref[...])
    u3 = u.reshape(N_TOKENS, T_BLK, N_C)
    glob = jnp.sum(u3[:, :, N_C // 2:], axis=0) / float(N_TOKENS)
    H = jnp.concatenate(
        [u3[:, :, :N_C // 2],
         jnp.broadcast_to(glob[None], (N_TOKENS, T_BLK, N_C // 2))], axis=-1)
    h2 = _gelu_pre(jnp.dot(H.reshape(NT, N_C), W2_ref[...],
                           preferred_element_type=jnp.float32) + b2_ref[...])
    h3 = _gelu_pre(jnp.dot(h2, W3_ref[...],
                           preferred_element_type=jnp.float32) + b3_ref[...])
    d = jnp.sum(h3 * w4d_ref[...], axis=-1).reshape(N_TOKENS, T_BLK)
    dt = d.T

    k = jax.lax.bitcast_convert_type(dt, jnp.int32)
    key = k ^ ((k >> 31) & jnp.int32(0x7FFFFFFF))
    ukey = key ^ jnp.int32(-2147483648)
    active = jnp.ones((T_BLK, N_TOKENS), jnp.int32)
    bottom = jnp.zeros((T_BLK, N_TOKENS), jnp.int32)
    need = jnp.full((T_BLK, 1), N_BOT, jnp.int32)
    for bit in range(31, -1, -1):
        bitv = (ukey >> bit) & 1
        zeros = active * (bitv ^ 1)
        nz = jnp.sum(zeros, axis=1, keepdims=True)
        go_zero = nz >= need
        bottom = jnp.where(go_zero, bottom, bottom + zeros)
        need = jnp.where(go_zero, need, need - nz)
        active = jnp.where(go_zero, zeros, active * bitv)
    idx = jax.lax.broadcasted_iota(jnp.int32, (T_BLK, N_TOKENS), 1)
    for bit in range(9, -1, -1):
        bitv = (idx >> bit) & 1
        ones = active * bitv
        n1 = jnp.sum(ones, axis=1, keepdims=True)
        go_one = n1 >= need
        bottom = jnp.where(go_one, bottom, bottom + ones)
        need = jnp.where(go_one, need, need - n1)
        active = jnp.where(go_one, ones, active * (bitv ^ 1))
    bottom = bottom + active * jnp.where(need > 0, 1, 0)

    bot_f = bottom.astype(jnp.float32)
    csum = jnp.dot(bot_f, tri_ref[...],
                   preferred_element_type=jnp.float32)
    kk = jax.lax.broadcasted_iota(jnp.int32, (T_BLK, N_TOKENS, N_BOT), 2)
    nn = jax.lax.broadcasted_iota(jnp.int32, (T_BLK, N_TOKENS, N_BOT), 1)
    csi = csum.astype(jnp.int32)[:, :, None]
    oh = ((csi == kk + 1) & (bottom[:, :, None] == 1)).astype(jnp.int32)
    tok = jnp.sum(oh * nn, axis=1)
    b = pl.program_id(0)
    tt = pl.program_id(1)
    rowc = (b * (N_TOKENS * N_T) + tt * T_BLK
            + jax.lax.broadcasted_iota(jnp.int32, (T_BLK, N_BOT), 0))
    flat = tok * N_T + rowc
    idx_ref[0] = jnp.concatenate(
        [flat, jnp.zeros((T_BLK, IDX_PAD - N_BOT), jnp.int32)], axis=1)
    sr_ref[0] = jnp.sum(X, axis=0)


_ROWS_PER_TILE = N_ROWS // 32


def _sc_gather_sum(idx2, xt):
    mesh = plsc.VectorSubcoreMesh(core_axis_name="c", subcore_axis_name="s")

    @functools.partial(
        pl.kernel, mesh=mesh,
        out_type=jax.ShapeDtypeStruct((N_ROWS, N_C), jnp.float32),
        scratch_types=[
            pltpu.VMEM((2, 128), jnp.int32),
            pltpu.VMEM((2, 8), jnp.int32),
            pltpu.VMEM((2, 128, N_C), jnp.float32),
            pltpu.VMEM((2, 8, N_C), jnp.float32),
            pltpu.VMEM((N_C,), jnp.float32),
            pltpu.SemaphoreType.DMA,
            pltpu.SemaphoreType.DMA,
        ],
    )
    def k(idx_hbm, xt_hbm, out_hbm, idxa_v, idxb_v, rows_a, rows_b,
          acc_v, sem0, sem1):
        wid = lax.axis_index("s") * 2 + lax.axis_index("c")
        sems = (sem0, sem1)
        pend = [None, None]

        def issue(rr, slot):
            r = wid * _ROWS_PER_TILE + rr
            pltpu.sync_copy(idx_hbm.at[r, pl.ds(0, 128)], idxa_v.at[slot])
            pltpu.sync_copy(idx_hbm.at[r, pl.ds(128, 8)], idxb_v.at[slot])
            ca = pltpu.async_copy(xt_hbm.at[idxa_v.at[slot]],
                                  rows_a.at[slot], sems[slot])
            cb = pltpu.async_copy(xt_hbm.at[idxb_v.at[slot]],
                                  rows_b.at[slot], sems[slot])
            pend[slot] = (ca, cb)

        issue(0, 0)
        for rr in range(_ROWS_PER_TILE):
            slot = rr % 2
            if rr + 1 < _ROWS_PER_TILE:
                issue(rr + 1, (rr + 1) % 2)
            ca, cb = pend[slot]
            ca.wait()
            cb.wait()

            def accum_a(j, acc, slot=slot):
                return tuple(acc[c] + rows_a[slot, j, pl.ds(c * 16, 16)]
                             for c in range(N_C // 16))

            acc = tuple(jnp.zeros((16,), jnp.float32)
                        for _ in range(N_C // 16))
            acc = lax.fori_loop(0, 128, accum_a, acc, unroll=False)

            def accum_b(j, acc, slot=slot):
                return tuple(acc[c] + rows_b[slot, j, pl.ds(c * 16, 16)]
                             for c in range(N_C // 16))

            acc = lax.fori_loop(0, N_BOT - 128, accum_b, acc, unroll=False)
            for c in range(N_C // 16):
                acc_v[pl.ds(c * 16, 16)] = acc[c]
            pltpu.sync_copy(acc_v, out_hbm.at[wid * _ROWS_PER_TILE + rr])

    return k(idx2, xt)


def _final_body(x_ref, sr_ref, bs_ref, wl_ref, o_ref):
    X = x_ref[0]
    wl2 = wl_ref[...]
    S = sr_ref[0] * wl2
    pooled = S / float(N_TOKENS)
    add2 = pooled + (S * (1.0 + wl2) - bs_ref[0] * wl2
                     - float(N_BOT) * pooled * wl2) / float(N_TOKENS - N_BOT)
    o_ref[0] = X + add2[None]


def kernel(x, wl, ln_g, ln_b, W1, b1, W2, b2, W3, b3, W4, b4):
    wl2 = wl.reshape(1, N_C)
    W1s = (ln_g[:, None] * W1) * _SQRT_HALF
    b1r = ((ln_b @ W1 + b1) * _SQRT_HALF).reshape(1, N_C)
    W2s = W2 * _SQRT_HALF
    b2r = (b2 * _SQRT_HALF).reshape(1, N_C // 2)
    W3s = W3 * _SQRT_HALF
    b3r = (b3 * _SQRT_HALF).reshape(1, N_C // 4)
    w4d = (W4[:, 0] - W4[:, 1]).reshape(1, N_C // 4)

    grid = (N_B, N_T // T_BLK)
    blk = pl.BlockSpec((1, N_TOKENS, T_BLK, N_C), lambda b, t: (b, 0, t, 0))
    small = lambda s: pl.BlockSpec(s, lambda b, t: (0,) * len(s))
    row_blk = lambda w: pl.BlockSpec((1, T_BLK, w), lambda b, t: (b, t, 0))

    ri = jnp.arange(N_TOKENS, dtype=jnp.int32)
    tri = (ri[:, None] <= ri[None, :]).astype(jnp.float32)

    idxs, sr = pl.pallas_call(
        _score_body,
        grid=grid,
        in_specs=[
            blk,
            small((N_C, N_C)), small((1, N_C)),
            small((N_C, N_C // 2)), small((1, N_C // 2)),
            small((N_C // 2, N_C // 4)), small((1, N_C // 4)),
            small((1, N_C // 4)),
            small((N_TOKENS, N_TOKENS)),
        ],
        out_specs=[row_blk(IDX_PAD), row_blk(N_C)],
        out_shape=[
            jax.ShapeDtypeStruct((N_B, N_T, IDX_PAD), jnp.int32),
            jax.ShapeDtypeStruct((N_B, N_T, N_C), jnp.float32),
        ],
        compiler_params=pltpu.CompilerParams(
            dimension_semantics=("parallel", "parallel")),
    )(x, W1s, b1r, W2s, b2r, W3s, b3r, w4d, tri)

    botsum = _sc_gather_sum(idxs.reshape(N_ROWS, IDX_PAD),
                            x.reshape(N_B * N_TOKENS * N_T, N_C))

    blk_c = pl.BlockSpec((1, N_TOKENS, N_T, N_C), lambda b: (b, 0, 0, 0))
    row_c = pl.BlockSpec((1, N_T, N_C), lambda b: (b, 0, 0))
    out = pl.pallas_call(
        _final_body,
        grid=(N_B,),
        in_specs=[blk_c, row_c, row_c, pl.BlockSpec((1, N_C), lambda b: (0, 0))],
        out_specs=blk_c,
        out_shape=jax.ShapeDtypeStruct((N_B, N_TOKENS, N_T, N_C), jnp.float32),
        compiler_params=pltpu.CompilerParams(
            dimension_semantics=("parallel",)),
    )(x, sr, botsum.reshape(N_B, N_T, N_C), wl2)
    return out
